# Initial kernel scaffold; baseline (speedup 1.0000x reference)
import jax
import jax.numpy as jnp
from jax.experimental import pallas as pl


def kernel(x_new_, y_points):
    raise NotImplementedError("write your pallas kernel here")


# speedup vs baseline: 3.2260x; 3.2260x over previous
import jax
import jax.numpy as jnp
from jax import lax
from jax.experimental import pallas as pl
from jax.experimental.pallas import tpu as pltpu
from jax.experimental.pallas import tpu_sc as plsc

B, N, Q = 2048, 8192, 2048
L = 16
NC, NS = 2, 16
NW = NC * NS
ROWS_PER_W = B // NW


def _tec_body(x_hbm, y_hbm, out_hbm, xv, idxv, wv, rowbuf, outbuf):
    wid = lax.axis_index("s") * NC + lax.axis_index("c")
    base_row = wid * ROWS_PER_W

    pltpu.sync_copy(x_hbm, xv)

    scale = jnp.float32(N - 1)

    def idx_body(i, carry):
        x = xv[pl.ds(i * L, L)]
        t = x * scale
        idx = t.astype(jnp.int32)
        idx = jnp.minimum(jnp.maximum(idx, 0), N - 2)
        w = t - idx.astype(jnp.float32)
        idxv[pl.ds(i * L, L)] = idx
        wv[pl.ds(i * L, L)] = w
        return carry

    lax.fori_loop(0, Q // L, idx_body, 0)

    def row_body(r, carry):
        row = base_row + r
        pltpu.sync_copy(y_hbm.at[row], rowbuf)

        def q_body(c, inner):
            iv = idxv[pl.ds(c * L, L)]
            w = wv[pl.ds(c * L, L)]
            y1 = plsc.load_gather(rowbuf, [iv])
            y2 = plsc.load_gather(rowbuf, [iv + 1])
            outbuf[pl.ds(c * L, L)] = y1 + w * (y2 - y1)
            return inner

        lax.fori_loop(0, Q // L, q_body, 0)
        pltpu.sync_copy(outbuf, out_hbm.at[row])
        return carry

    lax.fori_loop(0, ROWS_PER_W, row_body, 0)


def kernel(x_new_, y_points):
    mesh = plsc.VectorSubcoreMesh(core_axis_name="c", subcore_axis_name="s")
    k = pl.kernel(
        _tec_body,
        out_type=jax.ShapeDtypeStruct((B, Q), jnp.float32),
        mesh=mesh,
        compiler_params=pltpu.CompilerParams(needs_layout_passes=False),
        scratch_types=[
            pltpu.VMEM((Q,), jnp.float32),
            pltpu.VMEM((Q,), jnp.int32),
            pltpu.VMEM((Q,), jnp.float32),
            pltpu.VMEM((N,), jnp.float32),
            pltpu.VMEM((Q,), jnp.float32),
        ],
    )
    return k(x_new_, y_points)


# speedup vs baseline: 7.5690x; 2.3462x over previous
import jax
import jax.numpy as jnp
from jax import lax
from jax.experimental import pallas as pl
from jax.experimental.pallas import tpu as pltpu
from jax.experimental.pallas import tpu_sc as plsc

B, N, Q = 2048, 8192, 2048
L = 16
NC, NS = 2, 16
NW = NC * NS
ROWS_PER_W = B // NW
NBUF = 2


def _tec_body(x_hbm, y_hbm, out_hbm, xv, idxv, wv, rowbuf, outbuf,
              sem_in0, sem_in1, sem_out0, sem_out1):
    sems_in = (sem_in0, sem_in1)
    sems_out = (sem_out0, sem_out1)
    wid = lax.axis_index("s") * NC + lax.axis_index("c")
    base_row = wid * ROWS_PER_W

    pltpu.sync_copy(x_hbm, xv)

    scale = jnp.float32(N - 1)

    @plsc.parallel_loop(0, Q // L, unroll=4)
    def _idx_loop(i):
        x = xv[pl.ds(i * L, L)]
        t = x * scale
        idx = t.astype(jnp.int32)
        idx = jnp.minimum(jnp.maximum(idx, 0), N - 2)
        w = t - idx.astype(jnp.float32)
        idxv[pl.ds(i * L, L)] = idx
        wv[pl.ds(i * L, L)] = w

    def in_copy(b, row):
        return pltpu.make_async_copy(
            y_hbm.at[row], rowbuf.at[pl.ds(b * N, N)], sems_in[b])

    def out_copy(b, row):
        return pltpu.make_async_copy(
            outbuf.at[pl.ds(b * Q, Q)], out_hbm.at[row], sems_out[b])

    for b in range(NBUF):
        in_copy(b, base_row + b).start()

    def group_body(g, carry):
        for b in range(NBUF):
            r = g * NBUF + b
            row = base_row + r
            in_copy(b, row).wait()

            @pl.when(g > 0)
            def _():
                out_copy(b, row - NBUF).wait()

            boff = b * N

            @plsc.parallel_loop(0, Q // L, unroll=4)
            def _q_loop(c):
                iv = idxv[pl.ds(c * L, L)] + boff
                w = wv[pl.ds(c * L, L)]
                y1 = plsc.load_gather(rowbuf, [iv])
                y2 = plsc.load_gather(rowbuf, [iv + 1])
                outbuf[pl.ds(b * Q + c * L, L)] = y1 + w * (y2 - y1)

            out_copy(b, row).start()

            @pl.when(r + NBUF < ROWS_PER_W)
            def _():
                in_copy(b, row + NBUF).start()
        return carry

    lax.fori_loop(0, ROWS_PER_W // NBUF, group_body, 0)

    for b in range(NBUF):
        out_copy(b, base_row + ROWS_PER_W - NBUF + b).wait()


def kernel(x_new_, y_points):
    mesh = plsc.VectorSubcoreMesh(core_axis_name="c", subcore_axis_name="s")
    k = pl.kernel(
        _tec_body,
        out_type=jax.ShapeDtypeStruct((B, Q), jnp.float32),
        mesh=mesh,
        compiler_params=pltpu.CompilerParams(needs_layout_passes=False),
        scratch_types=[
            pltpu.VMEM((Q,), jnp.float32),
            pltpu.VMEM((Q,), jnp.int32),
            pltpu.VMEM((Q,), jnp.float32),
            pltpu.VMEM((NBUF * N,), jnp.float32),
            pltpu.VMEM((NBUF * Q,), jnp.float32),
            pltpu.SemaphoreType.DMA,
            pltpu.SemaphoreType.DMA,
            pltpu.SemaphoreType.DMA,
            pltpu.SemaphoreType.DMA,
        ],
    )
    return k(x_new_, y_points)


# speedup vs baseline: 10.1140x; 1.3362x over previous
import jax
import jax.numpy as jnp
from jax import lax
from jax.experimental import pallas as pl
from jax.experimental.pallas import tpu as pltpu
from jax.experimental.pallas import tpu_sc as plsc

B, N, Q = 2048, 8192, 2048
L = 16
NC, NS = 2, 16
NW = NC * NS
ROWS_PER_W = B // NW
NBUF = 4


def _tec_body(x_hbm, y_hbm, out_hbm, xv, idxv, wv, rowbuf, outbuf,
              sem_in0, sem_in1, sem_in2, sem_in3,
              sem_out0, sem_out1, sem_out2, sem_out3):
    sems_in = (sem_in0, sem_in1, sem_in2, sem_in3)
    sems_out = (sem_out0, sem_out1, sem_out2, sem_out3)
    wid = lax.axis_index("s") * NC + lax.axis_index("c")
    base_row = wid * ROWS_PER_W

    pltpu.sync_copy(x_hbm, xv)

    scale = jnp.float32(N - 1)

    @plsc.parallel_loop(0, Q // L, unroll=4)
    def _idx_loop(i):
        x = xv[pl.ds(i * L, L)]
        t = x * scale
        idx = t.astype(jnp.int32)
        idx = jnp.minimum(jnp.maximum(idx, 0), N - 2)
        w = t - idx.astype(jnp.float32)
        idxv[pl.ds(i * L, L)] = idx
        wv[pl.ds(i * L, L)] = w

    def in_copy(b, row):
        return pltpu.make_async_copy(
            y_hbm.at[row], rowbuf.at[pl.ds(b * N, N)], sems_in[b])

    def out_copy(b, row):
        return pltpu.make_async_copy(
            outbuf.at[pl.ds(b * Q, Q)], out_hbm.at[row], sems_out[b])

    for b in range(NBUF):
        in_copy(b, base_row + b).start()

    def group_body(g, carry):
        for b in range(NBUF):
            r = g * NBUF + b
            row = base_row + r
            in_copy(b, row).wait()

            @pl.when(g > 0)
            def _():
                out_copy(b, row - NBUF).wait()

            boff = b * N

            @plsc.parallel_loop(0, Q // L, unroll=4)
            def _q_loop(c):
                iv = idxv[pl.ds(c * L, L)] + boff
                w = wv[pl.ds(c * L, L)]
                y1 = plsc.load_gather(rowbuf, [iv])
                y2 = plsc.load_gather(rowbuf, [iv + 1])
                outbuf[pl.ds(b * Q + c * L, L)] = y1 + w * (y2 - y1)

            out_copy(b, row).start()

            @pl.when(r + NBUF < ROWS_PER_W)
            def _():
                in_copy(b, row + NBUF).start()
        return carry

    lax.fori_loop(0, ROWS_PER_W // NBUF, group_body, 0)

    for b in range(NBUF):
        out_copy(b, base_row + ROWS_PER_W - NBUF + b).wait()


def kernel(x_new_, y_points):
    mesh = plsc.VectorSubcoreMesh(core_axis_name="c", subcore_axis_name="s")
    k = pl.kernel(
        _tec_body,
        out_type=jax.ShapeDtypeStruct((B, Q), jnp.float32),
        mesh=mesh,
        compiler_params=pltpu.CompilerParams(needs_layout_passes=False),
        scratch_types=[
            pltpu.VMEM((Q,), jnp.float32),
            pltpu.VMEM((Q,), jnp.int32),
            pltpu.VMEM((Q,), jnp.float32),
            pltpu.VMEM((NBUF * N,), jnp.float32),
            pltpu.VMEM((NBUF * Q,), jnp.float32),
            pltpu.SemaphoreType.DMA,
            pltpu.SemaphoreType.DMA,
            pltpu.SemaphoreType.DMA,
            pltpu.SemaphoreType.DMA,
            pltpu.SemaphoreType.DMA,
            pltpu.SemaphoreType.DMA,
            pltpu.SemaphoreType.DMA,
            pltpu.SemaphoreType.DMA,
        ],
    )
    return k(x_new_, y_points)
